# Initial kernel scaffold; baseline (speedup 1.0000x reference)
import jax
import jax.numpy as jnp
from jax.experimental import pallas as pl


def kernel(h_frontier, neighbor_ids, rel_ids, type_ids, frontier_node_ids, type_emb, rel_emb, W_ih, W_hh, b_ih, b_hh, exp_W1, exp_b1, exp_w2, exp_b2, nbr_W1, nbr_b1, nbr_w2, nbr_b2):
    raise NotImplementedError("write your pallas kernel here")


# speedup vs baseline: 1.6127x; 1.6127x over previous
import functools

import jax
import jax.numpy as jnp
from jax import lax
from jax.experimental import pallas as pl
from jax.experimental.pallas import tpu as pltpu
from jax.experimental.pallas import tpu_sc as plsc

H = 128
NUM_NODES = 100000
NUM_REL = 12
F = 8192
DEG = 32
M = 2048
K = 16


_AT = 2048


def _bf(x):
    return x.astype(jnp.bfloat16).astype(jnp.float32)


def _dot_nt(a, b):
    return lax.dot_general(a.astype(jnp.bfloat16), b.astype(jnp.bfloat16),
                           (((1,), (1,)), ((), ())),
                           preferred_element_type=jnp.float32)


def _alpha_body(h_ref, w1_ref, b1_ref, w2_ref, out_ref):
    a = jax.nn.relu(_dot_nt(h_ref[...], w1_ref[...]) + b1_ref[...][None, :])
    out_ref[...] = jnp.sum(_bf(a) * _bf(w2_ref[...])[None, :], axis=1)


def _alpha(h_frontier, exp_W1, exp_b1, exp_w2):
    return pl.pallas_call(
        _alpha_body,
        grid=(F // _AT,),
        in_specs=[
            pl.BlockSpec((_AT, H), lambda i: (i, 0)),
            pl.BlockSpec((H, H), lambda i: (0, 0)),
            pl.BlockSpec((H,), lambda i: (0,)),
            pl.BlockSpec((H,), lambda i: (0,)),
        ],
        out_specs=pl.BlockSpec((_AT,), lambda i: (i,)),
        out_shape=jax.ShapeDtypeStruct((F,), jnp.float32),
    )(h_frontier, exp_W1, exp_b1, exp_w2)


_TM = 128
_ROWS = _TM * (K + 1)


def _edges_body(he_ref, rel_ref, nbr_ref, node_ref, type_ref,
                wih_ref, whh_ref, bih_ref, bhh_ref, relemb_ref, typeemb_ref,
                nw1_ref, nb1_ref, nw2_ref, msg_ref, dst_ref):
    he = he_ref[...]
    wih = wih_ref[...]
    bih = bih_ref[...]
    rel_gi = _dot_nt(relemb_ref[...], wih[:, :H])
    type_gi = _dot_nt(typeemb_ref[...], wih[:, H:])
    A = _dot_nt(he, whh_ref[...]) + bhh_ref[...][None, :]
    t_oh = (type_ref[...][:, None]
            == lax.broadcasted_iota(jnp.int32, (_TM, 4), 1)).astype(jnp.float32)
    T4 = jnp.sum(t_oh[:, :, None] * type_gi[None, :, :], axis=1)
    TB = T4 + bih[None, :]
    c_r = TB[:, :H] + A[:, :H]
    c_z = TB[:, H:2 * H] + A[:, H:2 * H]
    ti_n = TB[:, 2 * H:]
    h_n = A[:, 2 * H:]
    rg_r = rel_gi[:, :H][None, :, :]
    rg_z = rel_gi[:, H:2 * H][None, :, :]
    rg_n = rel_gi[:, 2 * H:][None, :, :]
    r_all = jax.nn.sigmoid(rg_r + c_r[:, None, :])
    z_all = jax.nn.sigmoid(rg_z + c_z[:, None, :])
    n_all = jnp.tanh(rg_n + ti_n[:, None, :] + r_all * h_n[:, None, :])
    g_all = (1.0 - z_all) * n_all + z_all * he[:, None, :]
    u = jax.nn.relu(
        _dot_nt(g_all.reshape(_TM * NUM_REL, H), nw1_ref[...])
        + nb1_ref[...][None, :])
    beta_all = jnp.sum(_bf(u) * _bf(nw2_ref[...])[None, :],
                       axis=1).reshape(_TM, NUM_REL)
    rel = rel_ref[...]
    oh3 = (rel[:, :, None]
           == lax.broadcasted_iota(jnp.int32, (_TM, DEG, NUM_REL), 2)).astype(jnp.float32)
    beta = jnp.sum(oh3 * beta_all[:, None, :], axis=2)
    nbr = nbr_ref[...]
    iota_e = lax.broadcasted_iota(jnp.int32, (_TM, DEG), 1)
    bw = beta
    g_ks = []
    d_ks = []
    for _ in range(K):
        mx = jnp.max(bw, axis=1, keepdims=True)
        cand = jnp.where(bw == mx, iota_e, DEG)
        j = jnp.min(cand, axis=1)
        ohk = iota_e == j[:, None]
        d_ks.append(jnp.sum(jnp.where(ohk, nbr, 0), axis=1))
        sel_rel = jnp.sum(oh3 * ohk.astype(jnp.float32)[:, :, None], axis=1)
        g_ks.append(jnp.sum(g_all * sel_rel[:, :, None], axis=1))
        bw = jnp.where(ohk, -1e30, bw)
    g_sel = jnp.stack(g_ks, axis=1)
    dst_sel = jnp.stack(d_ks, axis=1)
    msg_ref[...] = jnp.concatenate(
        [jnp.exp(g_sel).reshape(_TM * K, H), jnp.exp(he)], axis=0)
    dst_ref[...] = dst_sel


def _edges(hE, relE, nbrE, nodeE, typeE, W_ih, W_hh, b_ih, b_hh,
           rel_emb, type_emb, nbr_W1, nbr_b1, nbr_w2):
    full2 = lambda i: (0, 0)
    full1 = lambda i: (0,)
    return pl.pallas_call(
        _edges_body,
        grid=(M // _TM,),
        in_specs=[
            pl.BlockSpec((_TM, H), lambda i: (i, 0)),
            pl.BlockSpec((_TM, DEG), lambda i: (i, 0)),
            pl.BlockSpec((_TM, DEG), lambda i: (i, 0)),
            pl.BlockSpec((_TM,), lambda i: (i,)),
            pl.BlockSpec((_TM,), lambda i: (i,)),
            pl.BlockSpec((3 * H, 2 * H), full2),
            pl.BlockSpec((3 * H, H), full2),
            pl.BlockSpec((3 * H,), full1),
            pl.BlockSpec((3 * H,), full1),
            pl.BlockSpec((NUM_REL, H), full2),
            pl.BlockSpec((4, H), full2),
            pl.BlockSpec((H, H), full2),
            pl.BlockSpec((H,), full1),
            pl.BlockSpec((H,), full1),
        ],
        out_specs=[
            pl.BlockSpec((_ROWS, H), lambda i: (i, 0)),
            pl.BlockSpec((_TM, K), lambda i: (i, 0)),
        ],
        out_shape=[
            jax.ShapeDtypeStruct((M * (K + 1), H), jnp.float32),
            jax.ShapeDtypeStruct((M, K), jnp.int32),
        ],
    )(hE, relE, nbrE, nodeE, typeE, W_ih, W_hh, b_ih, b_hh,
      rel_emb, type_emb, nbr_W1, nbr_b1, nbr_w2)


NMSG = M * (K + 1)
CH = 12672
SP_ROWS = 12800
NCHUNK = 8
PAD_ROWS = NCHUNK * CH
NROUND = 4
_B = 128
_PER_SUB = NMSG // 16
_NBLK = _PER_SUB // _B
_ZR = 80


def _scatter_body(msg_hbm, dst_hbm, zeros_hbm, out_hbm,
                  shared, idx_v, msg_v, zero_v):
    c = lax.axis_index("c")
    s = lax.axis_index("s")
    pltpu.sync_copy(zeros_hbm, zero_v)

    def rnd_body(rnd, carry):
        chunk = rnd * 2 + c
        base = chunk * CH

        def zfill(z, cr):
            pltpu.sync_copy(zero_v,
                            shared.at[pl.ds(s * 800 + z * _ZR, _ZR), :])
            return cr

        lax.fori_loop(0, 10, zfill, 0)
        plsc.subcore_barrier()

        def blk(b, cr):
            off = s * _PER_SUB + b * _B
            pltpu.sync_copy(dst_hbm.at[pl.ds(off, _B)], idx_v)
            pltpu.sync_copy(msg_hbm.at[pl.ds(off, _B), :], msg_v)
            for k in range(_B // 16):
                iv = idx_v[pl.ds(k * 16, 16)]
                loc = iv - base
                ok = (loc >= 0) & (loc < CH)
                idx_v[pl.ds(k * 16, 16)] = jnp.where(ok, loc, CH)
            pltpu.sync_copy(msg_v, shared.at[idx_v], add=True)
            return cr

        lax.fori_loop(0, _NBLK, blk, 0)
        plsc.subcore_barrier()
        pltpu.sync_copy(shared.at[pl.ds(s * 792, 792), :],
                        out_hbm.at[pl.ds(base + s * 792, 792), :])
        plsc.subcore_barrier()
        return carry

    lax.fori_loop(0, NROUND, rnd_body, 0)


def _scatter(msgs, dst, zeros):
    mesh = plsc.VectorSubcoreMesh(core_axis_name="c", subcore_axis_name="s")
    fn = functools.partial(
        pl.kernel,
        mesh=mesh,
        out_type=jax.ShapeDtypeStruct((PAD_ROWS, H), jnp.float32),
        scratch_types=[
            pltpu.VMEM_SHARED((SP_ROWS, H), jnp.float32),
            pltpu.VMEM((_B,), jnp.int32),
            pltpu.VMEM((_B, H), jnp.float32),
            pltpu.VMEM((_ZR, H), jnp.float32),
        ],
    )(_scatter_body)
    return fn(msgs, dst, zeros)


_LT = 2000


def _finalize_body(s_ref, out_ref):
    sv = s_ref[...]
    out_ref[...] = jnp.where(sv > 0.0, jnp.log(jnp.maximum(sv, 1e-38)), 0.0)


def _finalize(sums_padded):
    return pl.pallas_call(
        _finalize_body,
        grid=(NUM_NODES // _LT,),
        in_specs=[pl.BlockSpec((_LT, H), lambda i: (i, 0))],
        out_specs=pl.BlockSpec((_LT, H), lambda i: (i, 0)),
        out_shape=jax.ShapeDtypeStruct((NUM_NODES, H), jnp.float32),
    )(sums_padded)


def kernel(h_frontier, neighbor_ids, rel_ids, type_ids, frontier_node_ids,
           type_emb, rel_emb, W_ih, W_hh, b_ih, b_hh,
           exp_W1, exp_b1, exp_w2, exp_b2,
           nbr_W1, nbr_b1, nbr_w2, nbr_b2):
    alpha = _alpha(h_frontier, exp_W1, exp_b1, exp_w2)
    _, top_idx = jax.lax.top_k(alpha, M)
    hE = jnp.take(h_frontier, top_idx, axis=0)
    nbrE = jnp.take(neighbor_ids, top_idx, axis=0)
    relE = jnp.take(rel_ids, top_idx, axis=0)
    typeE = jnp.take(type_ids, top_idx, axis=0)
    nodeE = jnp.take(frontier_node_ids, top_idx, axis=0)
    msgs, dst_e = _edges(hE, relE, nbrE, nodeE, typeE, W_ih, W_hh, b_ih, b_hh,
                         rel_emb, type_emb, nbr_W1, nbr_b1, nbr_w2)
    dst = jnp.concatenate(
        [dst_e.reshape(M // _TM, _TM * K), nodeE.reshape(M // _TM, _TM)],
        axis=1).reshape(NMSG)
    zeros = jnp.zeros((_ZR, H), dtype=jnp.float32)
    sums = _scatter(msgs, dst, zeros)
    return _finalize(sums)


# speedup vs baseline: 2.6497x; 1.6430x over previous
import functools

import jax
import jax.numpy as jnp
from jax import lax
from jax.experimental import pallas as pl
from jax.experimental.pallas import tpu as pltpu
from jax.experimental.pallas import tpu_sc as plsc

H = 128
NUM_NODES = 100000
NUM_REL = 12
F = 8192
DEG = 32
M = 2048
K = 16


_AT = 2048


def _bf(x):
    return x.astype(jnp.bfloat16).astype(jnp.float32)


def _dot_nt(a, b):
    return lax.dot_general(a.astype(jnp.bfloat16), b.astype(jnp.bfloat16),
                           (((1,), (1,)), ((), ())),
                           preferred_element_type=jnp.float32)


def _alpha_body(h_ref, w1_ref, b1_ref, w2_ref, out_ref):
    a = jax.nn.relu(_dot_nt(h_ref[...], w1_ref[...]) + b1_ref[...][None, :])
    out_ref[...] = jnp.sum(_bf(a) * _bf(w2_ref[...])[None, :], axis=1)


def _alpha(h_frontier, exp_W1, exp_b1, exp_w2):
    return pl.pallas_call(
        _alpha_body,
        grid=(F // _AT,),
        in_specs=[
            pl.BlockSpec((_AT, H), lambda i: (i, 0)),
            pl.BlockSpec((H, H), lambda i: (0, 0)),
            pl.BlockSpec((H,), lambda i: (0,)),
            pl.BlockSpec((H,), lambda i: (0,)),
        ],
        out_specs=pl.BlockSpec((_AT,), lambda i: (i,)),
        out_shape=jax.ShapeDtypeStruct((F,), jnp.float32),
    )(h_frontier, exp_W1, exp_b1, exp_w2)


_TM = 128
_ROWS = _TM * (K + 1)


def _edges_body(he_ref, rel_ref, nbr_ref, node_ref, type_ref,
                wih_ref, whh_ref, bih_ref, bhh_ref, relemb_ref, typeemb_ref,
                nw1_ref, nb1_ref, nw2_ref, msg_ref, dst_ref):
    he = he_ref[...]
    wih = wih_ref[...]
    bih = bih_ref[...]
    rel_gi = _dot_nt(relemb_ref[...], wih[:, :H])
    type_gi = _dot_nt(typeemb_ref[...], wih[:, H:])
    A = _dot_nt(he, whh_ref[...]) + bhh_ref[...][None, :]
    typ = type_ref[...]
    T4 = jnp.zeros((_TM, 3 * H), jnp.float32)
    for t in range(4):
        T4 = jnp.where(typ[:, None] == t, type_gi[t][None, :], T4)
    TB = T4 + bih[None, :]
    c_r = TB[:, :H] + A[:, :H]
    c_z = TB[:, H:2 * H] + A[:, H:2 * H]
    ti_n = TB[:, 2 * H:]
    h_n = A[:, 2 * H:]
    rg_r = rel_gi[:, :H][None, :, :]
    rg_z = rel_gi[:, H:2 * H][None, :, :]
    rg_n = rel_gi[:, 2 * H:][None, :, :]
    r_all = jax.nn.sigmoid(rg_r + c_r[:, None, :])
    z_all = jax.nn.sigmoid(rg_z + c_z[:, None, :])
    n_all = jnp.tanh(rg_n + ti_n[:, None, :] + r_all * h_n[:, None, :])
    g_all = (1.0 - z_all) * n_all + z_all * he[:, None, :]
    u = jax.nn.relu(
        _dot_nt(g_all.reshape(_TM * NUM_REL, H), nw1_ref[...])
        + nb1_ref[...][None, :])
    beta_all = jnp.sum(_bf(u) * _bf(nw2_ref[...])[None, :],
                       axis=1).reshape(_TM, NUM_REL)
    rel = rel_ref[...]
    beta = jnp.zeros((_TM, DEG), jnp.float32)
    for rr in range(NUM_REL):
        beta = jnp.where(rel == rr, beta_all[:, rr][:, None], beta)
    nbr = nbr_ref[...]
    iota_e = lax.broadcasted_iota(jnp.int32, (_TM, DEG), 1)
    bw = beta
    d_ks = []
    r_ks = []
    for _ in range(K):
        mx = jnp.max(bw, axis=1, keepdims=True)
        cand = jnp.where(bw == mx, iota_e, DEG)
        j = jnp.min(cand, axis=1)
        ohk = iota_e == j[:, None]
        d_ks.append(jnp.sum(jnp.where(ohk, nbr, 0), axis=1))
        r_ks.append(jnp.sum(jnp.where(ohk, rel, 0), axis=1))
        bw = jnp.where(ohk, -1e30, bw)
    dst_sel = jnp.stack(d_ks, axis=1)
    rel_sel = jnp.stack(r_ks, axis=1)
    g_sel = jnp.zeros((_TM, K, H), jnp.float32)
    for rr in range(NUM_REL):
        g_sel = jnp.where(rel_sel[:, :, None] == rr,
                          g_all[:, rr, :][:, None, :], g_sel)
    msg_ref[...] = jnp.concatenate(
        [jnp.exp(g_sel).reshape(_TM * K, H), jnp.exp(he)], axis=0)
    dst_ref[...] = dst_sel


def _edges(hE, relE, nbrE, nodeE, typeE, W_ih, W_hh, b_ih, b_hh,
           rel_emb, type_emb, nbr_W1, nbr_b1, nbr_w2):
    full2 = lambda i: (0, 0)
    full1 = lambda i: (0,)
    return pl.pallas_call(
        _edges_body,
        grid=(M // _TM,),
        in_specs=[
            pl.BlockSpec((_TM, H), lambda i: (i, 0)),
            pl.BlockSpec((_TM, DEG), lambda i: (i, 0)),
            pl.BlockSpec((_TM, DEG), lambda i: (i, 0)),
            pl.BlockSpec((_TM,), lambda i: (i,)),
            pl.BlockSpec((_TM,), lambda i: (i,)),
            pl.BlockSpec((3 * H, 2 * H), full2),
            pl.BlockSpec((3 * H, H), full2),
            pl.BlockSpec((3 * H,), full1),
            pl.BlockSpec((3 * H,), full1),
            pl.BlockSpec((NUM_REL, H), full2),
            pl.BlockSpec((4, H), full2),
            pl.BlockSpec((H, H), full2),
            pl.BlockSpec((H,), full1),
            pl.BlockSpec((H,), full1),
        ],
        out_specs=[
            pl.BlockSpec((_ROWS, H), lambda i: (i, 0)),
            pl.BlockSpec((_TM, K), lambda i: (i, 0)),
        ],
        out_shape=[
            jax.ShapeDtypeStruct((M * (K + 1), H), jnp.float32),
            jax.ShapeDtypeStruct((M, K), jnp.int32),
        ],
    )(hE, relE, nbrE, nodeE, typeE, W_ih, W_hh, b_ih, b_hh,
      rel_emb, type_emb, nbr_W1, nbr_b1, nbr_w2)


NMSG = M * (K + 1)
CH = 12672
SP_ROWS = 12800
NCHUNK = 8
PAD_ROWS = NCHUNK * CH
NROUND = 4
_B = 128
_PER_SUB = NMSG // 16
_NBLK = _PER_SUB // _B
_ZR = 80


def _scatter_body(msg_hbm, dst_hbm, zeros_hbm, out_hbm,
                  shared, idx_v, msg_v, zero_v):
    c = lax.axis_index("c")
    s = lax.axis_index("s")
    pltpu.sync_copy(zeros_hbm, zero_v)

    def rnd_body(rnd, carry):
        chunk = rnd * 2 + c
        base = chunk * CH

        def zfill(z, cr):
            pltpu.sync_copy(zero_v,
                            shared.at[pl.ds(s * 800 + z * _ZR, _ZR), :])
            return cr

        lax.fori_loop(0, 10, zfill, 0)
        plsc.subcore_barrier()

        def blk(b, cr):
            off = s * _PER_SUB + b * _B
            pltpu.sync_copy(dst_hbm.at[pl.ds(off, _B)], idx_v)
            pltpu.sync_copy(msg_hbm.at[pl.ds(off, _B), :], msg_v)
            for k in range(_B // 16):
                iv = idx_v[pl.ds(k * 16, 16)]
                loc = iv - base
                ok = (loc >= 0) & (loc < CH)
                idx_v[pl.ds(k * 16, 16)] = jnp.where(ok, loc, CH)
            pltpu.sync_copy(msg_v, shared.at[idx_v], add=True)
            return cr

        lax.fori_loop(0, _NBLK, blk, 0)
        plsc.subcore_barrier()
        pltpu.sync_copy(shared.at[pl.ds(s * 792, 792), :],
                        out_hbm.at[pl.ds(base + s * 792, 792), :])
        plsc.subcore_barrier()
        return carry

    lax.fori_loop(0, NROUND, rnd_body, 0)


def _scatter(msgs, dst, zeros):
    mesh = plsc.VectorSubcoreMesh(core_axis_name="c", subcore_axis_name="s")
    fn = functools.partial(
        pl.kernel,
        mesh=mesh,
        out_type=jax.ShapeDtypeStruct((PAD_ROWS, H), jnp.float32),
        scratch_types=[
            pltpu.VMEM_SHARED((SP_ROWS, H), jnp.float32),
            pltpu.VMEM((_B,), jnp.int32),
            pltpu.VMEM((_B, H), jnp.float32),
            pltpu.VMEM((_ZR, H), jnp.float32),
        ],
    )(_scatter_body)
    return fn(msgs, dst, zeros)


_LT = 2000


def _finalize_body(s_ref, out_ref):
    sv = s_ref[...]
    out_ref[...] = jnp.where(sv > 0.0, jnp.log(jnp.maximum(sv, 1e-38)), 0.0)


def _finalize(sums_padded):
    return pl.pallas_call(
        _finalize_body,
        grid=(NUM_NODES // _LT,),
        in_specs=[pl.BlockSpec((_LT, H), lambda i: (i, 0))],
        out_specs=pl.BlockSpec((_LT, H), lambda i: (i, 0)),
        out_shape=jax.ShapeDtypeStruct((NUM_NODES, H), jnp.float32),
    )(sums_padded)


def kernel(h_frontier, neighbor_ids, rel_ids, type_ids, frontier_node_ids,
           type_emb, rel_emb, W_ih, W_hh, b_ih, b_hh,
           exp_W1, exp_b1, exp_w2, exp_b2,
           nbr_W1, nbr_b1, nbr_w2, nbr_b2):
    alpha = _alpha(h_frontier, exp_W1, exp_b1, exp_w2)
    _, top_idx = jax.lax.top_k(alpha, M)
    hE = jnp.take(h_frontier, top_idx, axis=0)
    nbrE = jnp.take(neighbor_ids, top_idx, axis=0)
    relE = jnp.take(rel_ids, top_idx, axis=0)
    typeE = jnp.take(type_ids, top_idx, axis=0)
    nodeE = jnp.take(frontier_node_ids, top_idx, axis=0)
    msgs, dst_e = _edges(hE, relE, nbrE, nodeE, typeE, W_ih, W_hh, b_ih, b_hh,
                         rel_emb, type_emb, nbr_W1, nbr_b1, nbr_w2)
    dst = jnp.concatenate(
        [dst_e.reshape(M // _TM, _TM * K), nodeE.reshape(M // _TM, _TM)],
        axis=1).reshape(NMSG)
    zeros = jnp.zeros((_ZR, H), dtype=jnp.float32)
    sums = _scatter(msgs, dst, zeros)
    return _finalize(sums)
